# baseline (speedup 1.0000x reference)
import functools

import jax
import jax.numpy as jnp
from jax import lax
from jax.experimental import pallas as pl
from jax.experimental.pallas import tpu as pltpu
from jax.experimental.pallas import tpu_sc as plsc

BATCH = 4096
HIST = 200
EMB = 64

NC = 2
NS = 16
NW = NC * NS

B = BATCH * HIST
GW = 128
SUB = 4
CHUNK = GW * SUB
B_PER_W = B // NW
NCHUNK = B_PER_W // CHUNK
ROWS_PER_W = B_PER_W // GW


def _make_gather():
    mesh = plsc.VectorSubcoreMesh(
        core_axis_name="c", subcore_axis_name="s", num_cores=NC, num_subcores=NS
    )

    @functools.partial(
        pl.kernel,
        mesh=mesh,
        compiler_params=pltpu.CompilerParams(use_tc_tiling_on_sc=False),
        out_type=jax.ShapeDtypeStruct((B // GW, GW, EMB), jnp.float32),
        scratch_types=[
            pltpu.VMEM((ROWS_PER_W, GW), jnp.int32),
            pltpu.VMEM((2, SUB, GW, EMB), jnp.float32),
            pltpu.SemaphoreType.DMA,
            pltpu.SemaphoreType.DMA,
            pltpu.SemaphoreType.DMA,
        ],
    )
    def gather_kernel(idx_hbm, table_hbm, out_hbm, idx_v, rows_v, gsem, osem0, osem1):
        wid = lax.axis_index("s") * NC + lax.axis_index("c")
        row_base = wid * ROWS_PER_W
        pltpu.sync_copy(idx_hbm.at[pl.ds(row_base, ROWS_PER_W)], idx_v)

        osems = (osem0, osem1)

        def outer(go, _):
            for b in range(2):
                g = go * 2 + b
                @pl.when(go >= 1)
                def _():
                    pltpu.make_async_copy(
                        rows_v.at[b], out_hbm.at[pl.ds(row_base, SUB)], osems[b]
                    ).wait()

                descs = []
                for j in range(SUB):
                    descs.append(
                        pltpu.async_copy(
                            table_hbm.at[idx_v.at[g * SUB + j]],
                            rows_v.at[b, j],
                            gsem,
                        )
                    )
                for dsc in descs:
                    dsc.wait()
                pltpu.async_copy(
                    rows_v.at[b],
                    out_hbm.at[pl.ds(row_base + g * SUB, SUB)],
                    osems[b],
                )
            return 0

        lax.fori_loop(0, NCHUNK // 2, outer, 0)
        for b in range(2):
            pltpu.make_async_copy(
                rows_v.at[b], out_hbm.at[pl.ds(row_base, SUB)], osems[b]
            ).wait()

    return gather_kernel


_gather = _make_gather()


def kernel(x, table):
    idx = x.astype(jnp.int32).reshape(B // GW, GW)
    out = _gather(idx, table)
    return out.reshape(BATCH, HIST, EMB)


# speedup vs baseline: 1.0040x; 1.0040x over previous
import functools

import jax
import jax.numpy as jnp
from jax import lax
from jax.experimental import pallas as pl
from jax.experimental.pallas import tpu as pltpu
from jax.experimental.pallas import tpu_sc as plsc

BATCH = 4096
HIST = 200
EMB = 64

NC = 2
NS = 16
NW = NC * NS

B = BATCH * HIST
GW = 128
SUB = 4
CHUNK = GW * SUB
B_PER_W = B // NW
NCHUNK = B_PER_W // CHUNK
ROWS_PER_W = B_PER_W // GW


def _make_gather():
    mesh = plsc.VectorSubcoreMesh(
        core_axis_name="c", subcore_axis_name="s", num_cores=NC, num_subcores=NS
    )

    @functools.partial(
        pl.kernel,
        mesh=mesh,
        compiler_params=pltpu.CompilerParams(use_tc_tiling_on_sc=False),
        out_type=jax.ShapeDtypeStruct((B // GW, GW, EMB), jnp.float32),
        scratch_types=[
            pltpu.VMEM((ROWS_PER_W, GW), jnp.int32),
            pltpu.VMEM((2, SUB, GW, EMB), jnp.float32),
            pltpu.SemaphoreType.DMA,
            pltpu.SemaphoreType.DMA,
            pltpu.SemaphoreType.DMA,
            pltpu.SemaphoreType.DMA,
        ],
    )
    def gather_kernel(idx_hbm, table_hbm, out_hbm, idx_v, rows_v,
                      gsem0, gsem1, osem0, osem1):
        wid = lax.axis_index("s") * NC + lax.axis_index("c")
        row_base = wid * ROWS_PER_W
        pltpu.sync_copy(idx_hbm.at[pl.ds(row_base, ROWS_PER_W)], idx_v)

        gsems = (gsem0, gsem1)
        osems = (osem0, osem1)

        def start_gathers(g, b):
            for j in range(SUB):
                pltpu.async_copy(
                    table_hbm.at[idx_v.at[g * SUB + j]],
                    rows_v.at[b, j],
                    gsems[b],
                )

        def drain_gathers(b):
            for j in range(SUB):
                pltpu.make_async_copy(
                    table_hbm.at[idx_v.at[j]], rows_v.at[b, j], gsems[b]
                ).wait()

        def start_out(g, b):
            pltpu.async_copy(
                rows_v.at[b],
                out_hbm.at[pl.ds(row_base + g * SUB, SUB)],
                osems[b],
            )

        def wait_out(b):
            pltpu.make_async_copy(
                rows_v.at[b], out_hbm.at[pl.ds(row_base, SUB)], osems[b]
            ).wait()

        start_gathers(0, 0)

        def outer(go, _):
            @pl.when(go >= 1)
            def _():
                wait_out(1)
            start_gathers(go * 2 + 1, 1)
            drain_gathers(0)
            start_out(go * 2, 0)
            @pl.when(go < NCHUNK // 2 - 1)
            def _():
                wait_out(0)
                start_gathers(go * 2 + 2, 0)
            drain_gathers(1)
            start_out(go * 2 + 1, 1)
            return 0

        lax.fori_loop(0, NCHUNK // 2, outer, 0)
        wait_out(0)
        wait_out(1)

    return gather_kernel


_gather = _make_gather()


def kernel(x, table):
    idx = x.astype(jnp.int32).reshape(B // GW, GW)
    out = _gather(idx, table)
    return out.reshape(BATCH, HIST, EMB)
